# Initial kernel scaffold; baseline (speedup 1.0000x reference)
import jax
import jax.numpy as jnp
from jax.experimental import pallas as pl


def kernel(xy_pos, time, patch_rgb, edge_index, batch, W_vis, b_vis, time_emb, Wp1, bp1, Wp2, bp2, Wm1, bm1, Wm2, bm2, Wq, Wk, Wv, Wo, bo, Wf1, bf1, Wf2, bf2):
    raise NotImplementedError("write your pallas kernel here")


# speedup vs baseline: 1.0482x; 1.0482x over previous
import functools

import jax
import jax.numpy as jnp
from jax.experimental import pallas as pl

N = 10000
H = 8
DH = 32
BLK = 400


def _tail_body(feats_ref, comb_ref, Wf1_ref, bf1_ref, Wf2_ref, bf2_ref, out_ref):
    r = feats_ref[...] + comb_ref[...]
    h1 = jax.nn.gelu(r @ Wf1_ref[...] + bf1_ref[...])
    out_ref[...] = h1 @ Wf2_ref[...] + bf2_ref[...]


def kernel(xy_pos, time, patch_rgb, edge_index, batch, W_vis, b_vis, time_emb, Wp1, bp1, Wp2, bp2, Wm1, bm1, Wm2, bm2, Wq, Wk, Wv, Wo, bo, Wf1, bf1, Wf2, bf2):
    patch_feats = patch_rgb.reshape(patch_rgb.shape[0], -1) @ W_vis + b_vis
    time_feats = time_emb[time]
    pos_feats = jax.nn.gelu(xy_pos @ Wp1 + bp1) @ Wp2 + bp2
    combined = jnp.concatenate([patch_feats, pos_feats, time_feats], axis=-1)
    combined = jax.nn.gelu(combined @ Wm1 + bm1) @ Wm2 + bm2

    src = edge_index[0]
    dst = edge_index[1]
    q = (combined @ Wq).reshape(N, H, DH)
    k = (combined @ Wk).reshape(N, H, DH)
    v = (combined @ Wv).reshape(N, H, DH)
    scores = (q[dst] * k[src]).sum(-1) / jnp.sqrt(float(DH))
    ex = jnp.exp(scores)
    denom = jax.ops.segment_sum(ex, dst, num_segments=N)
    num = jax.ops.segment_sum(ex[:, :, None] * v[src], dst, num_segments=N)
    agg = num / (denom + 1e-16)[:, :, None]
    feats = agg.reshape(N, H * DH) @ Wo + bo

    grid = (N // BLK,)
    out = pl.pallas_call(
        _tail_body,
        grid=grid,
        in_specs=[
            pl.BlockSpec((BLK, 320), lambda i: (i, 0)),
            pl.BlockSpec((BLK, 320), lambda i: (i, 0)),
            pl.BlockSpec((320, 32), lambda i: (0, 0)),
            pl.BlockSpec((32,), lambda i: (0,)),
            pl.BlockSpec((32, 2), lambda i: (0, 0)),
            pl.BlockSpec((2,), lambda i: (0,)),
        ],
        out_specs=pl.BlockSpec((BLK, 2), lambda i: (i, 0)),
        out_shape=jax.ShapeDtypeStruct((N, 2), jnp.float32),
    )(feats, combined, Wf1, bf1, Wf2, bf2)
    return out


# speedup vs baseline: 12.8640x; 12.2725x over previous
import functools

import jax
import jax.numpy as jnp
from jax import lax
from jax.experimental import pallas as pl
from jax.experimental.pallas import tpu as pltpu
from jax.experimental.pallas import tpu_sc as plsc

N = 10000
NP = 10240
E = 160000
EP = 163840
H = 8
DH = 32
D = 256
BN = 256
BE = 128
NC = 2
NS = 16
ROWS_PER_SUB = NP // NS
BEA = 40
EPW_A = EP // (NC * NS)
NBLK_A = EPW_A // BEA
EPW_B = EP // NS
NBLK_B = EPW_B // BE
INV_SQRT_DH = 1.0 / (DH ** 0.5)


def _tc0_body(Wvis_ref, Wm1a_ref, Wm1b_ref, Wp2_ref, bvis_ref, bp2_ref, bm1_ref,
              Wfold_ref, Wp2m_ref, bias1_ref):
    Wfold_ref[...] = Wvis_ref[...] @ Wm1a_ref[...]
    Wp2m_ref[...] = Wp2_ref[...] @ Wm1b_ref[...]
    bias1_ref[...] = (bm1_ref[...] + bvis_ref[...] @ Wm1a_ref[...]
                      + bp2_ref[...] @ Wm1b_ref[...])


def _tc1_body(patch_ref, xy_ref, time_ref, temb_ref, Wfold_ref, Wp1_ref,
              bp1_ref, Wp2m_ref, WtM_ref, bias1_ref, Wm2_ref, bm2_ref,
              Wqp_ref, Wkp_ref, Wv_ref,
              comb_ref, qT_ref, kT_ref, v3_ref):
    a = patch_ref[...] @ Wfold_ref[...]
    a = a + jax.nn.gelu(xy_ref[...] @ Wp1_ref[...] + bp1_ref[...]) @ Wp2m_ref[...]
    t = time_ref[...][:, 0]
    oh = (t[:, None] == lax.broadcasted_iota(jnp.int32, (BN, 1000), 1)
          ).astype(jnp.float32)
    tf = oh @ temb_ref[...]
    a = a + tf @ WtM_ref[...] + bias1_ref[...]
    c2 = jax.nn.gelu(a) @ Wm2_ref[...] + bm2_ref[...]
    comb_ref[...] = c2
    qT_ref[...] = c2 @ Wqp_ref[...]
    kT_ref[...] = c2 @ Wkp_ref[...]
    v = c2 @ Wv_ref[...]
    v3_ref[0] = v[:, :128]
    v3_ref[1] = v[:, 128:]


def _sca_body(qT, kT, src, dst, exw_out, den_out,
              den_sp, isrc, idst, qrows, krows, exbuf, exwide, sem):
    c = lax.axis_index("c")
    s = lax.axis_index("s")
    wid = s * NC + c
    z16 = jnp.zeros((16,), jnp.float32)

    def zrow(r, carry):
        for j in range(8):
            exwide[r, pl.ds(16 * j, 16)] = z16
        return carry
    lax.fori_loop(0, BEA, zrow, 0)

    def zcp(i, carry):
        pltpu.sync_copy(exwide, den_sp.at[pl.ds(s * ROWS_PER_SUB + i * BEA, BEA)])
        return carry
    lax.fori_loop(0, ROWS_PER_SUB // BEA, zcp, 0)
    plsc.subcore_barrier()

    perm = lax.iota(jnp.int32, 16) ^ 8

    def blk_body(b, carry):
        base = wid * EPW_A + b * BEA
        pltpu.sync_copy(src.at[pl.ds(base, BEA)], isrc)
        pltpu.sync_copy(dst.at[pl.ds(base, BEA)], idst)
        pltpu.async_copy(qT.at[idst], qrows, sem).wait()
        pltpu.async_copy(kT.at[isrc], krows, sem).wait()

        def edge(e, ecarry):
            acc = qrows[e, pl.ds(0, 16)] * krows[e, pl.ds(0, 16)]
            for i in range(1, 16):
                acc = acc + qrows[e, pl.ds(16 * i, 16)] * krows[e, pl.ds(16 * i, 16)]
            s2 = acc + acc[perm]
            exv = jnp.exp(s2 * INV_SQRT_DH)
            exbuf[e, :] = exv
            exwide[e, pl.ds(0, 16)] = exv
            return ecarry
        lax.fori_loop(0, BEA, edge, 0)

        pltpu.sync_copy(exbuf, exw_out.at[pl.ds(base, BEA)])
        pltpu.sync_copy(exwide, den_sp.at[idst], add=True)
        return carry
    lax.fori_loop(0, NBLK_A, blk_body, 0)
    plsc.subcore_barrier()

    off = s * ROWS_PER_SUB
    pltpu.sync_copy(den_sp.at[pl.ds(off, ROWS_PER_SUB)],
                    den_out.at[c, pl.ds(off, ROWS_PER_SUB)])


def _scb_body(v2, src, dst, exw, num_out,
              num_sp, isrc, idst, isrc2, vrows, exv, sem):
    c = lax.axis_index("c")
    s = lax.axis_index("s")
    z16 = jnp.zeros((16,), jnp.float32)

    def zrow(r, carry):
        for j in range(8):
            vrows[r, pl.ds(16 * j, 16)] = z16
        return carry
    lax.fori_loop(0, BE, zrow, 0)

    def zcp(i, carry):
        pltpu.sync_copy(vrows, num_sp.at[pl.ds(s * ROWS_PER_SUB + i * BE, BE)])
        return carry
    lax.fori_loop(0, ROWS_PER_SUB // BE, zcp, 0)
    plsc.subcore_barrier()

    voff = c * NP
    hoff = c * 4
    idxv = [jnp.broadcast_to(hoff + t, (16,)).astype(jnp.int32) for t in range(4)]

    def blk_body(b, carry):
        base = s * EPW_B + b * BE
        pltpu.sync_copy(src.at[pl.ds(base, BE)], isrc)
        pltpu.sync_copy(dst.at[pl.ds(base, BE)], idst)

        def addo(i, icarry):
            isrc2[pl.ds(16 * i, 16)] = isrc[pl.ds(16 * i, 16)] + voff
            return icarry
        lax.fori_loop(0, BE // 16, addo, 0)

        pltpu.async_copy(v2.at[isrc2], vrows, sem).wait()
        pltpu.sync_copy(exw.at[pl.ds(base, BE)], exv)

        def edge(e, ecarry):
            exr = exv[e, :]
            g = [exr[idxv[t]] for t in range(4)]
            for j in range(8):
                vrows[e, pl.ds(16 * j, 16)] = vrows[e, pl.ds(16 * j, 16)] * g[j // 2]
            return ecarry
        lax.fori_loop(0, BE, edge, 0)

        pltpu.sync_copy(vrows, num_sp.at[idst], add=True)
        return carry
    lax.fori_loop(0, NBLK_B, blk_body, 0)
    plsc.subcore_barrier()

    off = s * ROWS_PER_SUB
    pltpu.sync_copy(num_sp.at[pl.ds(off, ROWS_PER_SUB)],
                    num_out.at[c, pl.ds(off, ROWS_PER_SUB)])


def _tc2_body(num_ref, den_ref, comb_ref, Wo_ref, bo_ref, Wf1_ref, bf1_ref,
              Wf2_ref, bf2_ref, out_ref):
    den8 = (den_ref[0, :, 0:8] + den_ref[1, :, 0:8]) + 1e-16
    hsel = (lax.broadcasted_iota(jnp.int32, (8, D), 0)
            == lax.broadcasted_iota(jnp.int32, (8, D), 1) // DH
            ).astype(jnp.float32)
    denx = den8 @ hsel
    agg = jnp.concatenate([num_ref[0], num_ref[1]], axis=1) / denx
    feats = agg @ Wo_ref[...] + bo_ref[...]
    r = feats + comb_ref[...]
    h1 = jax.nn.gelu(r @ Wf1_ref[...] + bf1_ref[...])
    out_ref[...] = h1 @ Wf2_ref[...] + bf2_ref[...]


def kernel(xy_pos, time, patch_rgb, edge_index, batch, W_vis, b_vis, time_emb,
           Wp1, bp1, Wp2, bp2, Wm1, bm1, Wm2, bm2, Wq, Wk, Wv, Wo, bo,
           Wf1, bf1, Wf2, bf2):
    f32 = jnp.float32
    patch_flat = jnp.pad(patch_rgb.reshape(N, -1), ((0, NP - N), (0, 0)))
    xy_p = jnp.pad(xy_pos, ((0, NP - N), (0, 0)))
    time_p = jnp.pad(time.astype(jnp.int32), (0, NP - N)).reshape(NP, 1)
    src_p = jnp.concatenate(
        [edge_index[0].astype(jnp.int32), jnp.full((EP - E,), N, jnp.int32)])
    dst_p = jnp.concatenate(
        [edge_index[1].astype(jnp.int32), jnp.full((EP - E,), N, jnp.int32)])
    Wqp = Wq.reshape(320, H, DH).transpose(0, 2, 1).reshape(320, D)
    Wkp = Wk.reshape(320, H, DH).transpose(0, 2, 1).reshape(320, D)
    Wm1a = Wm1[0:256]
    Wm1b = Wm1[256:288]
    WtM = Wm1[288:320]

    Wfold, Wp2m, bias1 = pl.pallas_call(
        _tc0_body,
        out_shape=(
            jax.ShapeDtypeStruct((768, 128), f32),
            jax.ShapeDtypeStruct((16, 128), f32),
            jax.ShapeDtypeStruct((128,), f32),
        ),
    )(W_vis, Wm1a, Wm1b, Wp2, b_vis, bp2, bm1)

    grid = (NP // BN,)
    comb, qT, kT, v3 = pl.pallas_call(
        _tc1_body,
        grid=grid,
        in_specs=[
            pl.BlockSpec((BN, 768), lambda i: (i, 0)),
            pl.BlockSpec((BN, 2), lambda i: (i, 0)),
            pl.BlockSpec((BN, 1), lambda i: (i, 0)),
            pl.BlockSpec((1000, 32), lambda i: (0, 0)),
            pl.BlockSpec((768, 128), lambda i: (0, 0)),
            pl.BlockSpec((2, 16), lambda i: (0, 0)),
            pl.BlockSpec((16,), lambda i: (0,)),
            pl.BlockSpec((16, 128), lambda i: (0, 0)),
            pl.BlockSpec((32, 128), lambda i: (0, 0)),
            pl.BlockSpec((128,), lambda i: (0,)),
            pl.BlockSpec((128, 320), lambda i: (0, 0)),
            pl.BlockSpec((320,), lambda i: (0,)),
            pl.BlockSpec((320, D), lambda i: (0, 0)),
            pl.BlockSpec((320, D), lambda i: (0, 0)),
            pl.BlockSpec((320, D), lambda i: (0, 0)),
        ],
        out_specs=(
            pl.BlockSpec((BN, 320), lambda i: (i, 0)),
            pl.BlockSpec((BN, D), lambda i: (i, 0)),
            pl.BlockSpec((BN, D), lambda i: (i, 0)),
            pl.BlockSpec((2, BN, 128), lambda i: (0, i, 0)),
        ),
        out_shape=(
            jax.ShapeDtypeStruct((NP, 320), f32),
            jax.ShapeDtypeStruct((NP, D), f32),
            jax.ShapeDtypeStruct((NP, D), f32),
            jax.ShapeDtypeStruct((2, NP, 128), f32),
        ),
    )(patch_flat, xy_p, time_p, time_emb, Wfold, Wp1, bp1, Wp2m, WtM, bias1,
      Wm2, bm2, Wqp, Wkp, Wv)

    v2 = v3.reshape(2 * NP, 128)

    mesh = plsc.VectorSubcoreMesh(core_axis_name="c", subcore_axis_name="s",
                                  num_cores=NC, num_subcores=NS)
    sca = functools.partial(
        pl.kernel,
        out_type=(
            jax.ShapeDtypeStruct((EP, 16), f32),
            jax.ShapeDtypeStruct((NC, NP, 128), f32),
        ),
        mesh=mesh,
        scratch_types=[
            pltpu.VMEM_SHARED((NP, 128), f32),
            pltpu.VMEM((BEA,), jnp.int32),
            pltpu.VMEM((BEA,), jnp.int32),
            pltpu.VMEM((BEA, D), f32),
            pltpu.VMEM((BEA, D), f32),
            pltpu.VMEM((BEA, 16), f32),
            pltpu.VMEM((BEA, 128), f32),
            pltpu.SemaphoreType.DMA,
        ],
    )(_sca_body)
    exw, den = sca(qT, kT, src_p, dst_p)

    scb = functools.partial(
        pl.kernel,
        out_type=jax.ShapeDtypeStruct((NC, NP, 128), f32),
        mesh=mesh,
        scratch_types=[
            pltpu.VMEM_SHARED((NP, 128), f32),
            pltpu.VMEM((BE,), jnp.int32),
            pltpu.VMEM((BE,), jnp.int32),
            pltpu.VMEM((BE,), jnp.int32),
            pltpu.VMEM((BE, 128), f32),
            pltpu.VMEM((BE, 16), f32),
            pltpu.SemaphoreType.DMA,
        ],
    )(_scb_body)
    num = scb(v2, src_p, dst_p, exw)

    out_full = pl.pallas_call(
        _tc2_body,
        grid=grid,
        in_specs=[
            pl.BlockSpec((2, BN, 128), lambda i: (0, i, 0)),
            pl.BlockSpec((2, BN, 128), lambda i: (0, i, 0)),
            pl.BlockSpec((BN, 320), lambda i: (i, 0)),
            pl.BlockSpec((D, 320), lambda i: (0, 0)),
            pl.BlockSpec((320,), lambda i: (0,)),
            pl.BlockSpec((320, 32), lambda i: (0, 0)),
            pl.BlockSpec((32,), lambda i: (0,)),
            pl.BlockSpec((32, 2), lambda i: (0, 0)),
            pl.BlockSpec((2,), lambda i: (0,)),
        ],
        out_specs=pl.BlockSpec((BN, 2), lambda i: (i, 0)),
        out_shape=jax.ShapeDtypeStruct((NP, 2), f32),
    )(num, den, comb, Wo, bo, Wf1, bf1, Wf2, bf2)

    return out_full[:N]


# speedup vs baseline: 21.7074x; 1.6875x over previous
import functools

import jax
import jax.numpy as jnp
from jax import lax
from jax.experimental import pallas as pl
from jax.experimental.pallas import tpu as pltpu
from jax.experimental.pallas import tpu_sc as plsc

N = 10000
NP = 10240
E = 160000
EP = 163840
H = 8
DH = 32
D = 256
BN = 256
BE = 80
NC = 2
NS = 16
ROWS_PER_SUB = NP // NS
BEA = 64
EPW_A = EP // (NC * NS)
NBLK_A = EPW_A // BEA
NDR = NP // 8
EPW_B = EP // NS
NBLK_B = EPW_B // BE
INV_SQRT_DH = 1.0 / (DH ** 0.5)


def _tc0_body(Wvis_ref, Wm1a_ref, Wm1b_ref, Wp2_ref, bvis_ref, bp2_ref, bm1_ref,
              Wfold_ref, Wp2m_ref, bias1_ref):
    Wfold_ref[...] = Wvis_ref[...] @ Wm1a_ref[...]
    Wp2m_ref[...] = Wp2_ref[...] @ Wm1b_ref[...]
    bias1_ref[...] = (bm1_ref[...] + bvis_ref[...] @ Wm1a_ref[...]
                      + bp2_ref[...] @ Wm1b_ref[...])


def _tc1_body(patch_ref, xy_ref, time_ref, temb_ref, Wfold_ref, Wp1_ref,
              bp1_ref, Wp2m_ref, WtM_ref, bias1_ref, Wm2_ref, bm2_ref,
              Wqp_ref, Wkp_ref, Wv_ref,
              comb_ref, qT_ref, kT_ref, v3_ref):
    a = patch_ref[...] @ Wfold_ref[...]
    a = a + jax.nn.gelu(xy_ref[...] @ Wp1_ref[...] + bp1_ref[...]) @ Wp2m_ref[...]
    t = time_ref[...][:, 0]
    oh = (t[:, None] == lax.broadcasted_iota(jnp.int32, (BN, 1000), 1)
          ).astype(jnp.float32)
    tf = oh @ temb_ref[...]
    a = a + tf @ WtM_ref[...] + bias1_ref[...]
    c2 = jax.nn.gelu(a) @ Wm2_ref[...] + bm2_ref[...]
    comb_ref[...] = c2
    qT_ref[...] = c2 @ Wqp_ref[...]
    kT_ref[...] = c2 @ Wkp_ref[...]
    v = c2 @ Wv_ref[...]
    v3_ref[0] = v[:, :128]
    v3_ref[1] = v[:, 128:]


def _sca_body(qT, kT, src, dst, exw_out, den_out, den_sp,
              isrc0, idst0, idr0, qrows0, krows0, exbuf0, exwide0,
              isrc1, idst1, idr1, qrows1, krows1, exbuf1, exwide1,
              semq0, semk0, sems0, seme0, semq1, semk1, sems1, seme1):
    c = lax.axis_index("c")
    s = lax.axis_index("s")
    wid = s * NC + c
    z16 = jnp.zeros((16,), jnp.float32)
    perm = lax.iota(jnp.int32, 16) ^ 8

    bufs = [
        (isrc0, idst0, idr0, qrows0, krows0, exbuf0, exwide0,
         semq0, semk0, sems0, seme0),
        (isrc1, idst1, idr1, qrows1, krows1, exbuf1, exwide1,
         semq1, semk1, sems1, seme1),
    ]

    def zrow(r, carry):
        for j in range(8):
            exwide0[r, pl.ds(16 * j, 16)] = z16
        return carry
    lax.fori_loop(0, BEA, zrow, 0)
    rps = NDR // NS

    def zcp(i, carry):
        pltpu.sync_copy(exwide0.at[pl.ds(0, 16)],
                        den_sp.at[pl.ds(s * rps + i * 16, 16)])
        return carry
    lax.fori_loop(0, rps // 16, zcp, 0)
    plsc.subcore_barrier()

    def fetch(buf, b):
        (isrc, idst, idr, qrows, krows, exbuf, exwide,
         semq, semk, sems, seme) = buf
        base = wid * EPW_A + b * BEA
        pltpu.sync_copy(src.at[pl.ds(base, BEA)], isrc)
        pltpu.sync_copy(dst.at[pl.ds(base, BEA)], idst)
        pltpu.async_copy(qT.at[idst], qrows, semq)
        pltpu.async_copy(kT.at[isrc], krows, semk)

    def process(buf, b, wait_prev):
        (isrc, idst, idr, qrows, krows, exbuf, exwide,
         semq, semk, sems, seme) = buf
        base = wid * EPW_A + b * BEA
        pltpu.make_async_copy(qT.at[idst], qrows, semq).wait()
        pltpu.make_async_copy(kT.at[isrc], krows, semk).wait()

        @pl.when(wait_prev)
        def _():
            pltpu.make_async_copy(exwide, den_sp.at[idr], sems).wait()
            pltpu.make_async_copy(exbuf, exw_out.at[pl.ds(base, BEA)], seme).wait()

        for g in range(BEA // 16):
            idr[pl.ds(16 * g, 16)] = lax.shift_right_logical(
                idst[pl.ds(16 * g, 16)], 3)

        def group(g, carry):
            idvec = idst[pl.ds(16 * g, 16)]
            for l in range(16):
                e = 16 * g + l
                col = (idvec[l] & 7) * 16
                acc = qrows[e, pl.ds(0, 16)] * krows[e, pl.ds(0, 16)]
                for i in range(1, 16):
                    acc = acc + qrows[e, pl.ds(16 * i, 16)] * krows[e, pl.ds(16 * i, 16)]
                s2 = acc + acc[perm]
                exv = jnp.exp(s2 * INV_SQRT_DH)
                exbuf[e, :] = exv
                for j in range(8):
                    exwide[e, pl.ds(16 * j, 16)] = z16
                exwide[e, pl.ds(col, 16)] = exv
            return carry
        lax.fori_loop(0, BEA // 16, group, 0)

        pltpu.async_copy(exbuf, exw_out.at[pl.ds(base, BEA)], seme)
        pltpu.async_copy(exwide, den_sp.at[idr], sems, add=True)

    fetch(bufs[0], 0)

    def pair(i, carry):
        b0 = 2 * i
        fetch(bufs[1], b0 + 1)
        process(bufs[0], b0, i > 0)

        @pl.when(i + 1 < NBLK_A // 2)
        def _():
            fetch(bufs[0], b0 + 2)
        process(bufs[1], b0 + 1, i > 0)
        return carry
    lax.fori_loop(0, NBLK_A // 2, pair, 0)

    pltpu.make_async_copy(exwide0, den_sp.at[idr0], sems0).wait()
    pltpu.make_async_copy(exbuf0, exw_out.at[pl.ds(0, BEA)], seme0).wait()
    pltpu.make_async_copy(exwide1, den_sp.at[idr1], sems1).wait()
    pltpu.make_async_copy(exbuf1, exw_out.at[pl.ds(0, BEA)], seme1).wait()
    plsc.subcore_barrier()

    off = s * rps
    pltpu.sync_copy(den_sp.at[pl.ds(off, rps)], den_out.at[c, pl.ds(off, rps)])


def _scb_body(v2, src, dst, exw, num_out, num_sp,
              isrc0, idst0, isrc20, vrows0, exv0,
              isrc1, idst1, isrc21, vrows1, exv1,
              semv0, semx0, sems0, semv1, semx1, sems1):
    c = lax.axis_index("c")
    s = lax.axis_index("s")
    z16 = jnp.zeros((16,), jnp.float32)

    bufs = [
        (isrc0, idst0, isrc20, vrows0, exv0, semv0, semx0, sems0),
        (isrc1, idst1, isrc21, vrows1, exv1, semv1, semx1, sems1),
    ]

    def zrow(r, carry):
        for j in range(8):
            vrows0[r, pl.ds(16 * j, 16)] = z16
        return carry
    lax.fori_loop(0, BE, zrow, 0)

    def zcp(i, carry):
        pltpu.sync_copy(vrows0, num_sp.at[pl.ds(s * ROWS_PER_SUB + i * BE, BE)])
        return carry
    lax.fori_loop(0, ROWS_PER_SUB // BE, zcp, 0)
    plsc.subcore_barrier()

    voff = c * NP
    hoff = c * 4
    idxv = [jnp.broadcast_to(hoff + t, (16,)).astype(jnp.int32) for t in range(4)]

    def fetch(buf, b, wait_sc):
        (isrc, idst, isrc2, vrows, exv, semv, semx, sems) = buf
        base = s * EPW_B + b * BE

        @pl.when(wait_sc)
        def _():
            pltpu.make_async_copy(vrows, num_sp.at[idst], sems).wait()

        pltpu.sync_copy(src.at[pl.ds(base, BE)], isrc)
        pltpu.sync_copy(dst.at[pl.ds(base, BE)], idst)
        for i in range(BE // 16):
            isrc2[pl.ds(16 * i, 16)] = isrc[pl.ds(16 * i, 16)] + voff
        pltpu.async_copy(v2.at[isrc2], vrows, semv)
        pltpu.async_copy(exw.at[pl.ds(base, BE)], exv, semx)

    def process(buf, b):
        (isrc, idst, isrc2, vrows, exv, semv, semx, sems) = buf
        pltpu.make_async_copy(v2.at[isrc2], vrows, semv).wait()
        pltpu.make_async_copy(exw.at[pl.ds(0, BE)], exv, semx).wait()

        def edge(e, ecarry):
            exr = exv[e, :]
            g = [exr[idxv[t]] for t in range(4)]
            for j in range(8):
                vrows[e, pl.ds(16 * j, 16)] = vrows[e, pl.ds(16 * j, 16)] * g[j // 2]
            return ecarry
        lax.fori_loop(0, BE, edge, 0)

        pltpu.async_copy(vrows, num_sp.at[idst], sems, add=True)

    fetch(bufs[0], 0, False)

    def pair(i, carry):
        b0 = 2 * i
        fetch(bufs[1], b0 + 1, i > 0)
        process(bufs[0], b0)

        @pl.when(i + 1 < NBLK_B // 2)
        def _():
            fetch(bufs[0], b0 + 2, True)
        process(bufs[1], b0 + 1)
        return carry
    lax.fori_loop(0, NBLK_B // 2, pair, 0)

    pltpu.make_async_copy(vrows0, num_sp.at[idst0], sems0).wait()
    pltpu.make_async_copy(vrows1, num_sp.at[idst1], sems1).wait()
    plsc.subcore_barrier()

    off = s * ROWS_PER_SUB
    pltpu.sync_copy(num_sp.at[pl.ds(off, ROWS_PER_SUB)],
                    num_out.at[c, pl.ds(off, ROWS_PER_SUB)])


def _tc2_body(num_ref, den_ref, comb_ref, Wo_ref, bo_ref, Wf1_ref, bf1_ref,
              Wf2_ref, bf2_ref, out_ref):
    den8 = (den_ref[0, :, 0:8] + den_ref[1, :, 0:8]) + 1e-16
    hsel = (lax.broadcasted_iota(jnp.int32, (8, D), 0)
            == lax.broadcasted_iota(jnp.int32, (8, D), 1) // DH
            ).astype(jnp.float32)
    denx = den8 @ hsel
    agg = jnp.concatenate([num_ref[0], num_ref[1]], axis=1) / denx
    feats = agg @ Wo_ref[...] + bo_ref[...]
    r = feats + comb_ref[...]
    h1 = jax.nn.gelu(r @ Wf1_ref[...] + bf1_ref[...])
    out_ref[...] = h1 @ Wf2_ref[...] + bf2_ref[...]


def kernel(xy_pos, time, patch_rgb, edge_index, batch, W_vis, b_vis, time_emb,
           Wp1, bp1, Wp2, bp2, Wm1, bm1, Wm2, bm2, Wq, Wk, Wv, Wo, bo,
           Wf1, bf1, Wf2, bf2):
    f32 = jnp.float32
    patch_flat = jnp.pad(patch_rgb.reshape(N, -1), ((0, NP - N), (0, 0)))
    xy_p = jnp.pad(xy_pos, ((0, NP - N), (0, 0)))
    time_p = jnp.pad(time.astype(jnp.int32), (0, NP - N)).reshape(NP, 1)
    src_p = jnp.concatenate(
        [edge_index[0].astype(jnp.int32), jnp.full((EP - E,), N, jnp.int32)])
    dst_p = jnp.concatenate(
        [edge_index[1].astype(jnp.int32), jnp.full((EP - E,), N, jnp.int32)])
    Wqp = Wq.reshape(320, H, DH).transpose(0, 2, 1).reshape(320, D)
    Wkp = Wk.reshape(320, H, DH).transpose(0, 2, 1).reshape(320, D)
    Wm1a = Wm1[0:256]
    Wm1b = Wm1[256:288]
    WtM = Wm1[288:320]

    Wfold, Wp2m, bias1 = pl.pallas_call(
        _tc0_body,
        out_shape=(
            jax.ShapeDtypeStruct((768, 128), f32),
            jax.ShapeDtypeStruct((16, 128), f32),
            jax.ShapeDtypeStruct((128,), f32),
        ),
    )(W_vis, Wm1a, Wm1b, Wp2, b_vis, bp2, bm1)

    grid = (NP // BN,)
    comb, qT, kT, v3 = pl.pallas_call(
        _tc1_body,
        grid=grid,
        in_specs=[
            pl.BlockSpec((BN, 768), lambda i: (i, 0)),
            pl.BlockSpec((BN, 2), lambda i: (i, 0)),
            pl.BlockSpec((BN, 1), lambda i: (i, 0)),
            pl.BlockSpec((1000, 32), lambda i: (0, 0)),
            pl.BlockSpec((768, 128), lambda i: (0, 0)),
            pl.BlockSpec((2, 16), lambda i: (0, 0)),
            pl.BlockSpec((16,), lambda i: (0,)),
            pl.BlockSpec((16, 128), lambda i: (0, 0)),
            pl.BlockSpec((32, 128), lambda i: (0, 0)),
            pl.BlockSpec((128,), lambda i: (0,)),
            pl.BlockSpec((128, 320), lambda i: (0, 0)),
            pl.BlockSpec((320,), lambda i: (0,)),
            pl.BlockSpec((320, D), lambda i: (0, 0)),
            pl.BlockSpec((320, D), lambda i: (0, 0)),
            pl.BlockSpec((320, D), lambda i: (0, 0)),
        ],
        out_specs=(
            pl.BlockSpec((BN, 320), lambda i: (i, 0)),
            pl.BlockSpec((BN, D), lambda i: (i, 0)),
            pl.BlockSpec((BN, D), lambda i: (i, 0)),
            pl.BlockSpec((2, BN, 128), lambda i: (0, i, 0)),
        ),
        out_shape=(
            jax.ShapeDtypeStruct((NP, 320), f32),
            jax.ShapeDtypeStruct((NP, D), f32),
            jax.ShapeDtypeStruct((NP, D), f32),
            jax.ShapeDtypeStruct((2, NP, 128), f32),
        ),
    )(patch_flat, xy_p, time_p, time_emb, Wfold, Wp1, bp1, Wp2m, WtM, bias1,
      Wm2, bm2, Wqp, Wkp, Wv)

    v2 = v3.reshape(2 * NP, 128)

    mesh = plsc.VectorSubcoreMesh(core_axis_name="c", subcore_axis_name="s",
                                  num_cores=NC, num_subcores=NS)
    sca = functools.partial(
        pl.kernel,
        out_type=(
            jax.ShapeDtypeStruct((EP, 16), f32),
            jax.ShapeDtypeStruct((NC, NDR, 128), f32),
        ),
        mesh=mesh,
        scratch_types=[
            pltpu.VMEM_SHARED((NDR, 128), f32),
        ] + 2 * [
            pltpu.VMEM((BEA,), jnp.int32),
            pltpu.VMEM((BEA,), jnp.int32),
            pltpu.VMEM((BEA,), jnp.int32),
            pltpu.VMEM((BEA, D), f32),
            pltpu.VMEM((BEA, D), f32),
            pltpu.VMEM((BEA, 16), f32),
            pltpu.VMEM((BEA, 128), f32),
        ] + 8 * [pltpu.SemaphoreType.DMA],
    )(_sca_body)
    exw, den = sca(qT, kT, src_p, dst_p)
    den = den.reshape(NC, NP, 16)

    scb = functools.partial(
        pl.kernel,
        out_type=jax.ShapeDtypeStruct((NC, NP, 128), f32),
        mesh=mesh,
        scratch_types=[
            pltpu.VMEM_SHARED((NP, 128), f32),
        ] + 2 * [
            pltpu.VMEM((BE,), jnp.int32),
            pltpu.VMEM((BE,), jnp.int32),
            pltpu.VMEM((BE,), jnp.int32),
            pltpu.VMEM((BE, 128), f32),
            pltpu.VMEM((BE, 16), f32),
        ] + 6 * [pltpu.SemaphoreType.DMA],
    )(_scb_body)
    num = scb(v2, src_p, dst_p, exw)

    out_full = pl.pallas_call(
        _tc2_body,
        grid=grid,
        in_specs=[
            pl.BlockSpec((2, BN, 128), lambda i: (0, i, 0)),
            pl.BlockSpec((2, BN, 16), lambda i: (0, i, 0)),
            pl.BlockSpec((BN, 320), lambda i: (i, 0)),
            pl.BlockSpec((D, 320), lambda i: (0, 0)),
            pl.BlockSpec((320,), lambda i: (0,)),
            pl.BlockSpec((320, 32), lambda i: (0, 0)),
            pl.BlockSpec((32,), lambda i: (0,)),
            pl.BlockSpec((32, 2), lambda i: (0, 0)),
            pl.BlockSpec((2,), lambda i: (0,)),
        ],
        out_specs=pl.BlockSpec((BN, 2), lambda i: (i, 0)),
        out_shape=jax.ShapeDtypeStruct((NP, 2), f32),
    )(num, den, comb, Wo, bo, Wf1, bf1, Wf2, bf2)

    return out_full[:N]


# speedup vs baseline: 22.6488x; 1.0434x over previous
import functools

import jax
import jax.numpy as jnp
from jax import lax
from jax.experimental import pallas as pl
from jax.experimental.pallas import tpu as pltpu
from jax.experimental.pallas import tpu_sc as plsc

N = 10000
NP = 10240
E = 160000
EP = 163840
H = 8
DH = 32
D = 256
BN = 256
BE = 64
NC = 2
NS = 16
ROWS_PER_SUB = NP // NS
BEA = 64
EPW_A = EP // (NC * NS)
NBLK_A = EPW_A // BEA
NDR = NP // 8
EPW_B = EP // NS
NBLK_B = EPW_B // BE
INV_SQRT_DH = 1.0 / (DH ** 0.5)


def _tc0_body(Wvis_ref, Wm1a_ref, Wm1b_ref, Wp2_ref, bvis_ref, bp2_ref, bm1_ref,
              Wfold_ref, Wp2m_ref, bias1_ref):
    Wfold_ref[...] = Wvis_ref[...] @ Wm1a_ref[...]
    Wp2m_ref[...] = Wp2_ref[...] @ Wm1b_ref[...]
    bias1_ref[...] = (bm1_ref[...] + bvis_ref[...] @ Wm1a_ref[...]
                      + bp2_ref[...] @ Wm1b_ref[...])


def _tc1_body(patch_ref, xy_ref, time_ref, temb_ref, Wfold_ref, Wp1_ref,
              bp1_ref, Wp2m_ref, WtM_ref, bias1_ref, Wm2_ref, bm2_ref,
              Wqp_ref, Wkp_ref, Wv_ref,
              comb_ref, qT_ref, kT_ref, v3_ref):
    a = patch_ref[...] @ Wfold_ref[...]
    a = a + jax.nn.gelu(xy_ref[...] @ Wp1_ref[...] + bp1_ref[...]) @ Wp2m_ref[...]
    t = time_ref[...][:, 0]
    oh = (t[:, None] == lax.broadcasted_iota(jnp.int32, (BN, 1000), 1)
          ).astype(jnp.float32)
    tf = oh @ temb_ref[...]
    a = a + tf @ WtM_ref[...] + bias1_ref[...]
    c2 = jax.nn.gelu(a) @ Wm2_ref[...] + bm2_ref[...]
    comb_ref[...] = c2
    qT_ref[...] = c2 @ Wqp_ref[...]
    kT_ref[...] = c2 @ Wkp_ref[...]
    v = c2 @ Wv_ref[...]
    v3_ref[0] = v[:, :128]
    v3_ref[1] = v[:, 128:]


def _sca_body(qT, kT, src, dst, exw_out, den_out, den_sp,
              isrc_all, idst_all,
              idr0, qrows0, krows0, exbuf0, exwide0,
              idr1, qrows1, krows1, exbuf1, exwide1,
              semq0, semk0, sems0, seme0, semq1, semk1, sems1, seme1):
    c = lax.axis_index("c")
    s = lax.axis_index("s")
    wid = s * NC + c
    z16 = jnp.zeros((16,), jnp.float32)
    perm = lax.iota(jnp.int32, 16) ^ 8

    bufs = [
        (idr0, qrows0, krows0, exbuf0, exwide0, semq0, semk0, sems0, seme0),
        (idr1, qrows1, krows1, exbuf1, exwide1, semq1, semk1, sems1, seme1),
    ]

    ebase = wid * EPW_A
    pltpu.sync_copy(src.at[pl.ds(ebase, EPW_A)], isrc_all)
    pltpu.sync_copy(dst.at[pl.ds(ebase, EPW_A)], idst_all)

    def zrow(r, carry):
        for j in range(8):
            exwide0[r, pl.ds(16 * j, 16)] = z16
        return carry
    lax.fori_loop(0, BEA, zrow, 0)
    rps = NDR // NS

    def zcp(i, carry):
        pltpu.sync_copy(exwide0.at[pl.ds(0, 16)],
                        den_sp.at[pl.ds(s * rps + i * 16, 16)])
        return carry
    lax.fori_loop(0, rps // 16, zcp, 0)
    plsc.subcore_barrier()

    def fetch(buf, b):
        (idr, qrows, krows, exbuf, exwide, semq, semk, sems, seme) = buf
        boff = b * BEA
        pltpu.async_copy(qT.at[idst_all.at[pl.ds(boff, BEA)]], qrows, semq)
        pltpu.async_copy(kT.at[isrc_all.at[pl.ds(boff, BEA)]], krows, semk)

    def process(buf, b, wait_prev):
        (idr, qrows, krows, exbuf, exwide, semq, semk, sems, seme) = buf
        base = wid * EPW_A + b * BEA
        boff = b * BEA
        pltpu.make_async_copy(qT.at[idst_all.at[pl.ds(boff, BEA)]], qrows, semq).wait()
        pltpu.make_async_copy(kT.at[isrc_all.at[pl.ds(boff, BEA)]], krows, semk).wait()

        @pl.when(wait_prev)
        def _():
            pltpu.make_async_copy(exwide, den_sp.at[idr], sems).wait()
            pltpu.make_async_copy(exbuf, exw_out.at[pl.ds(base, BEA)], seme).wait()

        for g in range(BEA // 16):
            idr[pl.ds(16 * g, 16)] = lax.shift_right_logical(
                idst_all[pl.ds(boff + 16 * g, 16)], 3)

        def group(g, carry):
            idvec = idst_all[pl.ds(boff + 16 * g, 16)]
            for l in range(16):
                e = 16 * g + l
                col = (idvec[l] & 7) * 16
                acc = qrows[e, pl.ds(0, 16)] * krows[e, pl.ds(0, 16)]
                for i in range(1, 16):
                    acc = acc + qrows[e, pl.ds(16 * i, 16)] * krows[e, pl.ds(16 * i, 16)]
                s2 = acc + acc[perm]
                exv = jnp.exp(s2 * INV_SQRT_DH)
                exbuf[e, :] = exv
                for j in range(8):
                    exwide[e, pl.ds(16 * j, 16)] = z16
                exwide[e, pl.ds(col, 16)] = exv
            return carry
        lax.fori_loop(0, BEA // 16, group, 0)

        pltpu.async_copy(exbuf, exw_out.at[pl.ds(base, BEA)], seme)
        pltpu.async_copy(exwide, den_sp.at[idr], sems, add=True)

    fetch(bufs[0], 0)

    def pair(i, carry):
        b0 = 2 * i
        fetch(bufs[1], b0 + 1)
        process(bufs[0], b0, i > 0)

        @pl.when(i + 1 < NBLK_A // 2)
        def _():
            fetch(bufs[0], b0 + 2)
        process(bufs[1], b0 + 1, i > 0)
        return carry
    lax.fori_loop(0, NBLK_A // 2, pair, 0)

    pltpu.make_async_copy(exwide0, den_sp.at[idr0], sems0).wait()
    pltpu.make_async_copy(exbuf0, exw_out.at[pl.ds(0, BEA)], seme0).wait()
    pltpu.make_async_copy(exwide1, den_sp.at[idr1], sems1).wait()
    pltpu.make_async_copy(exbuf1, exw_out.at[pl.ds(0, BEA)], seme1).wait()
    plsc.subcore_barrier()

    off = s * rps
    pltpu.sync_copy(den_sp.at[pl.ds(off, rps)], den_out.at[c, pl.ds(off, rps)])


def _scb_body(v2, src, dst, exw, num_out, num_sp, isrc_all,
              idst0, vrows0, exv0,
              idst1, vrows1, exv1,
              semv0, semx0, sems0, semv1, semx1, sems1):
    c = lax.axis_index("c")
    s = lax.axis_index("s")
    z16 = jnp.zeros((16,), jnp.float32)

    bufs = [
        (idst0, vrows0, exv0, semv0, semx0, sems0),
        (idst1, vrows1, exv1, semv1, semx1, sems1),
    ]

    ebase = s * EPW_B
    pltpu.sync_copy(src.at[pl.ds(ebase, EPW_B)], isrc_all)
    voff0 = c * NP

    def addo(i, carry):
        isrc_all[pl.ds(16 * i, 16)] = isrc_all[pl.ds(16 * i, 16)] + voff0
        return carry
    lax.fori_loop(0, EPW_B // 16, addo, 0)

    def zrow(r, carry):
        for j in range(8):
            vrows0[r, pl.ds(16 * j, 16)] = z16
        return carry
    lax.fori_loop(0, BE, zrow, 0)

    def zcp(i, carry):
        pltpu.sync_copy(vrows0, num_sp.at[pl.ds(s * ROWS_PER_SUB + i * BE, BE)])
        return carry
    lax.fori_loop(0, ROWS_PER_SUB // BE, zcp, 0)
    plsc.subcore_barrier()

    hoff = c * 4
    idxv = [jnp.broadcast_to(hoff + t, (16,)).astype(jnp.int32) for t in range(4)]

    def fetch(buf, b, wait_sc):
        (idst, vrows, exv, semv, semx, sems) = buf
        base = s * EPW_B + b * BE
        boff = b * BE

        @pl.when(wait_sc)
        def _():
            pltpu.make_async_copy(vrows, num_sp.at[idst], sems).wait()

        pltpu.sync_copy(dst.at[pl.ds(base, BE)], idst)
        pltpu.async_copy(v2.at[isrc_all.at[pl.ds(boff, BE)]], vrows, semv)
        pltpu.async_copy(exw.at[pl.ds(base, BE)], exv, semx)

    def process(buf, b):
        (idst, vrows, exv, semv, semx, sems) = buf
        boff = b * BE
        pltpu.make_async_copy(v2.at[isrc_all.at[pl.ds(boff, BE)]], vrows, semv).wait()
        pltpu.make_async_copy(exw.at[pl.ds(0, BE)], exv, semx).wait()

        def edge(e, ecarry):
            exr = exv[e, :]
            g = [exr[idxv[t]] for t in range(4)]
            for j in range(8):
                vrows[e, pl.ds(16 * j, 16)] = vrows[e, pl.ds(16 * j, 16)] * g[j // 2]
            return ecarry
        lax.fori_loop(0, BE, edge, 0)

        pltpu.async_copy(vrows, num_sp.at[idst], sems, add=True)

    fetch(bufs[0], 0, False)

    def pair(i, carry):
        b0 = 2 * i
        fetch(bufs[1], b0 + 1, i > 0)
        process(bufs[0], b0)

        @pl.when(i + 1 < NBLK_B // 2)
        def _():
            fetch(bufs[0], b0 + 2, True)
        process(bufs[1], b0 + 1)
        return carry
    lax.fori_loop(0, NBLK_B // 2, pair, 0)

    pltpu.make_async_copy(vrows0, num_sp.at[idst0], sems0).wait()
    pltpu.make_async_copy(vrows1, num_sp.at[idst1], sems1).wait()
    plsc.subcore_barrier()

    off = s * ROWS_PER_SUB
    pltpu.sync_copy(num_sp.at[pl.ds(off, ROWS_PER_SUB)],
                    num_out.at[c, pl.ds(off, ROWS_PER_SUB)])


def _tc2_body(num_ref, den_ref, comb_ref, Wo_ref, bo_ref, Wf1_ref, bf1_ref,
              Wf2_ref, bf2_ref, out_ref):
    den8 = (den_ref[0, :, 0:8] + den_ref[1, :, 0:8]) + 1e-16
    hsel = (lax.broadcasted_iota(jnp.int32, (8, D), 0)
            == lax.broadcasted_iota(jnp.int32, (8, D), 1) // DH
            ).astype(jnp.float32)
    denx = den8 @ hsel
    agg = jnp.concatenate([num_ref[0], num_ref[1]], axis=1) / denx
    feats = agg @ Wo_ref[...] + bo_ref[...]
    r = feats + comb_ref[...]
    h1 = jax.nn.gelu(r @ Wf1_ref[...] + bf1_ref[...])
    out_ref[...] = h1 @ Wf2_ref[...] + bf2_ref[...]


def kernel(xy_pos, time, patch_rgb, edge_index, batch, W_vis, b_vis, time_emb,
           Wp1, bp1, Wp2, bp2, Wm1, bm1, Wm2, bm2, Wq, Wk, Wv, Wo, bo,
           Wf1, bf1, Wf2, bf2):
    f32 = jnp.float32
    patch_flat = jnp.pad(patch_rgb.reshape(N, -1), ((0, NP - N), (0, 0)))
    xy_p = jnp.pad(xy_pos, ((0, NP - N), (0, 0)))
    time_p = jnp.pad(time.astype(jnp.int32), (0, NP - N)).reshape(NP, 1)
    src_p = jnp.concatenate(
        [edge_index[0].astype(jnp.int32), jnp.full((EP - E,), N, jnp.int32)])
    dst_p = jnp.concatenate(
        [edge_index[1].astype(jnp.int32), jnp.full((EP - E,), N, jnp.int32)])
    Wqp = Wq.reshape(320, H, DH).transpose(0, 2, 1).reshape(320, D)
    Wkp = Wk.reshape(320, H, DH).transpose(0, 2, 1).reshape(320, D)
    Wm1a = Wm1[0:256]
    Wm1b = Wm1[256:288]
    WtM = Wm1[288:320]

    Wfold, Wp2m, bias1 = pl.pallas_call(
        _tc0_body,
        out_shape=(
            jax.ShapeDtypeStruct((768, 128), f32),
            jax.ShapeDtypeStruct((16, 128), f32),
            jax.ShapeDtypeStruct((128,), f32),
        ),
    )(W_vis, Wm1a, Wm1b, Wp2, b_vis, bp2, bm1)

    grid = (NP // BN,)
    comb, qT, kT, v3 = pl.pallas_call(
        _tc1_body,
        grid=grid,
        in_specs=[
            pl.BlockSpec((BN, 768), lambda i: (i, 0)),
            pl.BlockSpec((BN, 2), lambda i: (i, 0)),
            pl.BlockSpec((BN, 1), lambda i: (i, 0)),
            pl.BlockSpec((1000, 32), lambda i: (0, 0)),
            pl.BlockSpec((768, 128), lambda i: (0, 0)),
            pl.BlockSpec((2, 16), lambda i: (0, 0)),
            pl.BlockSpec((16,), lambda i: (0,)),
            pl.BlockSpec((16, 128), lambda i: (0, 0)),
            pl.BlockSpec((32, 128), lambda i: (0, 0)),
            pl.BlockSpec((128,), lambda i: (0,)),
            pl.BlockSpec((128, 320), lambda i: (0, 0)),
            pl.BlockSpec((320,), lambda i: (0,)),
            pl.BlockSpec((320, D), lambda i: (0, 0)),
            pl.BlockSpec((320, D), lambda i: (0, 0)),
            pl.BlockSpec((320, D), lambda i: (0, 0)),
        ],
        out_specs=(
            pl.BlockSpec((BN, 320), lambda i: (i, 0)),
            pl.BlockSpec((BN, D), lambda i: (i, 0)),
            pl.BlockSpec((BN, D), lambda i: (i, 0)),
            pl.BlockSpec((2, BN, 128), lambda i: (0, i, 0)),
        ),
        out_shape=(
            jax.ShapeDtypeStruct((NP, 320), f32),
            jax.ShapeDtypeStruct((NP, D), f32),
            jax.ShapeDtypeStruct((NP, D), f32),
            jax.ShapeDtypeStruct((2, NP, 128), f32),
        ),
    )(patch_flat, xy_p, time_p, time_emb, Wfold, Wp1, bp1, Wp2m, WtM, bias1,
      Wm2, bm2, Wqp, Wkp, Wv)

    v2 = v3.reshape(2 * NP, 128)

    mesh = plsc.VectorSubcoreMesh(core_axis_name="c", subcore_axis_name="s",
                                  num_cores=NC, num_subcores=NS)
    sca = functools.partial(
        pl.kernel,
        out_type=(
            jax.ShapeDtypeStruct((EP, 16), f32),
            jax.ShapeDtypeStruct((NC, NDR, 128), f32),
        ),
        mesh=mesh,
        scratch_types=[
            pltpu.VMEM_SHARED((NDR, 128), f32),
            pltpu.VMEM((EPW_A,), jnp.int32),
            pltpu.VMEM((EPW_A,), jnp.int32),
        ] + 2 * [
            pltpu.VMEM((BEA,), jnp.int32),
            pltpu.VMEM((BEA, D), f32),
            pltpu.VMEM((BEA, D), f32),
            pltpu.VMEM((BEA, 16), f32),
            pltpu.VMEM((BEA, 128), f32),
        ] + 8 * [pltpu.SemaphoreType.DMA],
    )(_sca_body)
    exw, den = sca(qT, kT, src_p, dst_p)
    den = den.reshape(NC, NP, 16)

    scb = functools.partial(
        pl.kernel,
        out_type=jax.ShapeDtypeStruct((NC, NP, 128), f32),
        mesh=mesh,
        scratch_types=[
            pltpu.VMEM_SHARED((NP, 128), f32),
            pltpu.VMEM((EPW_B,), jnp.int32),
        ] + 2 * [
            pltpu.VMEM((BE,), jnp.int32),
            pltpu.VMEM((BE, 128), f32),
            pltpu.VMEM((BE, 16), f32),
        ] + 6 * [pltpu.SemaphoreType.DMA],
    )(_scb_body)
    num = scb(v2, src_p, dst_p, exw)

    out_full = pl.pallas_call(
        _tc2_body,
        grid=grid,
        in_specs=[
            pl.BlockSpec((2, BN, 128), lambda i: (0, i, 0)),
            pl.BlockSpec((2, BN, 16), lambda i: (0, i, 0)),
            pl.BlockSpec((BN, 320), lambda i: (i, 0)),
            pl.BlockSpec((D, 320), lambda i: (0, 0)),
            pl.BlockSpec((320,), lambda i: (0,)),
            pl.BlockSpec((320, 32), lambda i: (0, 0)),
            pl.BlockSpec((32,), lambda i: (0,)),
            pl.BlockSpec((32, 2), lambda i: (0, 0)),
            pl.BlockSpec((2,), lambda i: (0,)),
        ],
        out_specs=pl.BlockSpec((BN, 2), lambda i: (i, 0)),
        out_shape=jax.ShapeDtypeStruct((NP, 2), f32),
    )(num, den, comb, Wo, bo, Wf1, bf1, Wf2, bf2)

    return out_full[:N]
